# Initial kernel scaffold; baseline (speedup 1.0000x reference)
import jax
import jax.numpy as jnp
from jax.experimental import pallas as pl


def kernel(code_levels, table_0, table_1, table_2, table_3):
    raise NotImplementedError("write your pallas kernel here")


# speedup vs baseline: 4.6251x; 4.6251x over previous
import jax
import jax.numpy as jnp
from jax import lax
from jax.experimental import pallas as pl
from jax.experimental.pallas import tpu as pltpu
from jax.experimental.pallas import tpu_sc as plsc

_B = 100000
_DIMS = (16, 32, 32, 48)
_OFFS = (0, 16, 48, 80)
_OUT_D = 128
_NC, _NS = 2, 16
_NW = _NC * _NS
_SG = 128
_GPC = 2
_C = _SG * _GPC
_NSUB = -(-_B // _SG)
_BPAD = _NSUB * _SG
_K = _NSUB // _GPC
_TAIL = _B - (_K - 1) * _C


def _body(cl3, t0, t1, t2, t3, out, idx_v, r0, r1, r2, r3, gsem, wsem):
    tabs = (t0, t1, t2, t3)
    rows = (r0, r1, r2, r3)
    wid = lax.axis_index("s") * _NC + lax.axis_index("c")
    nk = (_K - 1 - wid) // _NW + 1

    def step(i, carry):
        k = wid + i * _NW
        s = pl.multiple_of(k * _C, _C)
        pltpu.sync_copy(cl3.at[:, pl.ds(_GPC * k, _GPC), :], idx_v)
        gcps = [
            pltpu.async_copy(tabs[l].at[idx_v.at[l, j]],
                             rows[l].at[pl.ds(j * _SG, _SG)], gsem)
            for l in range(4) for j in range(_GPC)
        ]
        for cp in gcps:
            cp.wait()

        @pl.when(k != _K - 1)
        def _full_write():
            wcps = [
                pltpu.async_copy(
                    rows[l],
                    out.at[pl.ds(s, _C), pl.ds(_OFFS[l], _DIMS[l])], wsem)
                for l in range(4)
            ]
            for cp in wcps:
                cp.wait()

        @pl.when(k == _K - 1)
        def _tail_write():
            wcps = [
                pltpu.async_copy(
                    rows[l].at[pl.ds(0, _TAIL)],
                    out.at[pl.ds(s, _TAIL), pl.ds(_OFFS[l], _DIMS[l])], wsem)
                for l in range(4)
            ]
            for cp in wcps:
                cp.wait()

        return carry

    lax.fori_loop(0, nk, step, 0)


@jax.jit
def kernel(code_levels, table_0, table_1, table_2, table_3):
    cl_t = code_levels.T.astype(jnp.int32)
    cl3 = jnp.pad(cl_t, ((0, 0), (0, _BPAD - _B))).reshape(4, _NSUB, _SG)
    run = pl.kernel(
        _body,
        out_type=jax.ShapeDtypeStruct((_B, _OUT_D), jnp.float32),
        mesh=plsc.VectorSubcoreMesh(core_axis_name="c", subcore_axis_name="s",
                                    num_cores=_NC, num_subcores=_NS),
        scratch_types=[
            pltpu.VMEM((4, _GPC, _SG), jnp.int32),
            pltpu.VMEM((_C, _DIMS[0]), jnp.float32),
            pltpu.VMEM((_C, _DIMS[1]), jnp.float32),
            pltpu.VMEM((_C, _DIMS[2]), jnp.float32),
            pltpu.VMEM((_C, _DIMS[3]), jnp.float32),
            pltpu.SemaphoreType.DMA,
            pltpu.SemaphoreType.DMA,
        ],
        compiler_params=pltpu.CompilerParams(use_tc_tiling_on_sc=False),
    )
    return run(cl3, table_0, table_1, table_2, table_3)


# speedup vs baseline: 4.6463x; 1.0046x over previous
import jax
import jax.numpy as jnp
from jax import lax
from jax.experimental import pallas as pl
from jax.experimental.pallas import tpu as pltpu
from jax.experimental.pallas import tpu_sc as plsc

_B = 100000
_DIMS = (16, 32, 32, 48)
_OFFS = (0, 16, 48, 80)
_OUT_D = 128
_NC, _NS = 2, 16
_NW = _NC * _NS
_SG = 128
_GPC = 2
_C = _SG * _GPC
_NSUB = -(-_B // _SG)
_BPAD = _NSUB * _SG
_K = _B // _C
_TAIL = _B - _K * _C
_NKMAX = -(-_K // _NW)
_PMAX = -(-_NKMAX // 2)
_TAILW = _NW - 1


def _body(cl3, t0, t1, t2, t3, out,
          idx0, idx1, a0, a1, a2, a3, b0, b1, b2, b3,
          gsem, isem0, isem1, wsem0, wsem1):
    tabs = (t0, t1, t2, t3)
    rows = ((a0, a1, a2, a3), (b0, b1, b2, b3))
    idxs = (idx0, idx1)
    isems = (isem0, isem1)
    wsems = (wsem0, wsem1)
    wid = lax.axis_index("s") * _NC + lax.axis_index("c")
    nk = (_K - 1 - wid) // _NW + 1

    def out_slc(s, l):
        return out.at[pl.ds(s, _C), pl.ds(_OFFS[l], _DIMS[l])]

    k0 = wid
    pltpu.async_copy(cl3.at[:, pl.ds(_GPC * k0, _GPC), :], idxs[0], isems[0])

    def chunk(i, b):
        k = wid + i * _NW
        s = pl.multiple_of(k * _C, _C)

        @pl.when(i >= 2)
        def _drain():
            for l in range(4):
                pltpu.make_async_copy(rows[b][l], out_slc(s, l),
                                      wsems[b]).wait()

        pltpu.make_async_copy(cl3.at[:, pl.ds(_GPC * k, _GPC), :],
                              idxs[b], isems[b]).wait()

        gcps = [
            pltpu.async_copy(tabs[l].at[idxs[b].at[l, j]],
                             rows[b][l].at[pl.ds(j * _SG, _SG)], gsem)
            for l in range(4) for j in range(_GPC)
        ]

        @pl.when(i + 1 < nk)
        def _prefetch():
            kn = k + _NW
            pltpu.async_copy(cl3.at[:, pl.ds(_GPC * kn, _GPC), :],
                             idxs[1 - b], isems[1 - b])

        for cp in gcps:
            cp.wait()

        for l in range(4):
            pltpu.async_copy(rows[b][l], out_slc(s, l), wsems[b])

    def pair(p, carry):
        for b in (0, 1):
            i = 2 * p + b

            @pl.when(i < nk)
            def _():
                chunk(i, b)

        return carry

    lax.fori_loop(0, _PMAX, pair, 0)

    for b in (0, 1):
        @pl.when(nk > b)
        def _():
            for l in range(4):
                pltpu.make_async_copy(rows[b][l], out_slc(0, l),
                                      wsems[b]).wait()

    @pl.when(wid == _TAILW)
    def _tail():
        pltpu.sync_copy(cl3.at[:, pl.ds(_GPC * _K, _GPC), :], idxs[0])
        gcps = [
            pltpu.async_copy(tabs[l].at[idxs[0].at[l, j]],
                             rows[0][l].at[pl.ds(j * _SG, _SG)], gsem)
            for l in range(4) for j in range(_GPC)
        ]
        for cp in gcps:
            cp.wait()
        wcps = [
            pltpu.async_copy(
                rows[0][l].at[pl.ds(0, _TAIL)],
                out.at[pl.ds(_K * _C, _TAIL), pl.ds(_OFFS[l], _DIMS[l])],
                wsems[0])
            for l in range(4)
        ]
        for cp in wcps:
            cp.wait()


@jax.jit
def kernel(code_levels, table_0, table_1, table_2, table_3):
    cl_t = code_levels.T.astype(jnp.int32)
    cl3 = jnp.pad(cl_t, ((0, 0), (0, _BPAD - _B))).reshape(4, _NSUB, _SG)
    run = pl.kernel(
        _body,
        out_type=jax.ShapeDtypeStruct((_B, _OUT_D), jnp.float32),
        mesh=plsc.VectorSubcoreMesh(core_axis_name="c", subcore_axis_name="s",
                                    num_cores=_NC, num_subcores=_NS),
        scratch_types=[
            pltpu.VMEM((4, _GPC, _SG), jnp.int32),
            pltpu.VMEM((4, _GPC, _SG), jnp.int32),
            pltpu.VMEM((_C, _DIMS[0]), jnp.float32),
            pltpu.VMEM((_C, _DIMS[1]), jnp.float32),
            pltpu.VMEM((_C, _DIMS[2]), jnp.float32),
            pltpu.VMEM((_C, _DIMS[3]), jnp.float32),
            pltpu.VMEM((_C, _DIMS[0]), jnp.float32),
            pltpu.VMEM((_C, _DIMS[1]), jnp.float32),
            pltpu.VMEM((_C, _DIMS[2]), jnp.float32),
            pltpu.VMEM((_C, _DIMS[3]), jnp.float32),
            pltpu.SemaphoreType.DMA,
            pltpu.SemaphoreType.DMA,
            pltpu.SemaphoreType.DMA,
            pltpu.SemaphoreType.DMA,
            pltpu.SemaphoreType.DMA,
        ],
        compiler_params=pltpu.CompilerParams(use_tc_tiling_on_sc=False),
    )
    return run(cl3, table_0, table_1, table_2, table_3)
